# baseline (speedup 1.0000x reference)
import functools

import jax
import jax.numpy as jnp
from jax.experimental import pallas as pl
from jax.experimental.pallas import tpu as pltpu

N = 32768
IN_DIM = 1024
HD = 256
K1 = 8
BLK = 1024
NBLK = N // BLK
ROWS = N // 128

_PREC = jax.lax.Precision.HIGHEST


def _dot(a, b):
    return jax.lax.dot_general(a, b, (((1,), (0,)), ((), ())),
                               precision=_PREC,
                               preferred_element_type=jnp.float32)


def _fwd_body(h_ref, wfc_ref, bfc_ref,
              wa1_ref, ba1_ref, wb1_ref, bb1_ref, wc1_ref, bc1_ref,
              wa2_ref, ba2_ref, wb2_ref, bb2_ref, wc2_ref, bc2_ref,
              x_ref, a1_ref, a2_ref):
    x = jnp.maximum(_dot(h_ref[:], wfc_ref[:]) + bfc_ref[:], 0.0)
    x_ref[:] = x

    def gated(wa, ba, wb, bb, wc, bc):
        a = jnp.tanh(_dot(x, wa) + ba)
        b = jax.nn.sigmoid(_dot(x, wb) + bb)
        return _dot(a * b, wc) + bc

    a1_ref[:] = gated(wa1_ref[:], ba1_ref[:], wb1_ref[:], bb1_ref[:],
                      wc1_ref[:], bc1_ref[:])
    a2_ref[:] = gated(wa2_ref[:], ba2_ref[:], wb2_ref[:], bb2_ref[:],
                      wc2_ref[:], bc2_ref[:])


def _sel_body(a1_ref, a2_ref, cx_ref, cy_ref, w_ref):
    rows = jax.lax.broadcasted_iota(jnp.int32, (ROWS, 128), 0)
    cols = jax.lax.broadcasted_iota(jnp.int32, (ROWS, 128), 1)
    flat = rows * 128 + cols

    a1 = a1_ref[:]
    cx = cx_ref[:]
    cy = cy_ref[:]
    big = jnp.float32(1e30)
    xmin = big
    xmax = -big
    ymin = big
    ymax = -big
    for _ in range(K1):
        m = jnp.max(a1)
        sidx = jnp.min(jnp.where(a1 >= m, flat, jnp.int32(2**30)))
        sel = flat == sidx
        xmin = jnp.minimum(xmin, jnp.min(jnp.where(sel, cx, big)))
        xmax = jnp.maximum(xmax, jnp.max(jnp.where(sel, cx, -big)))
        ymin = jnp.minimum(ymin, jnp.min(jnp.where(sel, cy, big)))
        ymax = jnp.maximum(ymax, jnp.max(jnp.where(sel, cy, -big)))
        a1 = jnp.where(sel, -big, a1)

    inb = ((cx >= xmin) & (cx <= xmax) & (cy >= ymin) & (cy <= ymax))
    a2 = a2_ref[:]
    mx = jnp.max(jnp.where(inb, a2, -big))
    e = jnp.where(inb, jnp.exp(a2 - mx), 0.0)
    z = jnp.sum(e)
    w_ref[:] = e / z


def _pool_body(x_ref, w_ref, wcls_ref, bcls_ref, out_ref, acc_ref):
    i = pl.program_id(0)

    @pl.when(i == 0)
    def _():
        acc_ref[:] = jnp.zeros_like(acc_ref)

    acc_ref[:] += jnp.sum(x_ref[:] * w_ref[:], axis=0, keepdims=True)

    @pl.when(i == NBLK - 1)
    def _():
        out_ref[:] = _dot(acc_ref[:], wcls_ref[:]) + bcls_ref[:]


@jax.jit
def kernel(h, coords, W_fc, b_fc, Wa1, ba1, Wb1, bb1, Wc1, bc1,
           Wa2, ba2, Wb2, bb2, Wc2, bc2, W_cls, b_cls):
    f32 = jnp.float32
    wspec = pl.BlockSpec((IN_DIM, HD), lambda i: (0, 0))
    hspec = pl.BlockSpec((HD, HD), lambda i: (0, 0))
    vspec = pl.BlockSpec((1, HD), lambda i: (0, 0))
    cspec = pl.BlockSpec((HD, 1), lambda i: (0, 0))
    sspec = pl.BlockSpec((1, 1), lambda i: (0, 0))

    x, a1, a2 = pl.pallas_call(
        _fwd_body,
        grid=(NBLK,),
        in_specs=[
            pl.BlockSpec((BLK, IN_DIM), lambda i: (i, 0)),
            wspec, vspec,
            hspec, vspec, hspec, vspec, cspec, sspec,
            hspec, vspec, hspec, vspec, cspec, sspec,
        ],
        out_specs=(
            pl.BlockSpec((BLK, HD), lambda i: (i, 0)),
            pl.BlockSpec((BLK, 1), lambda i: (i, 0)),
            pl.BlockSpec((BLK, 1), lambda i: (i, 0)),
        ),
        out_shape=(
            jax.ShapeDtypeStruct((N, HD), f32),
            jax.ShapeDtypeStruct((N, 1), f32),
            jax.ShapeDtypeStruct((N, 1), f32),
        ),
        compiler_params=pltpu.CompilerParams(
            dimension_semantics=("parallel",)),
    )(h, W_fc, b_fc.reshape(1, HD),
      Wa1, ba1.reshape(1, HD), Wb1, bb1.reshape(1, HD), Wc1,
      bc1.reshape(1, 1),
      Wa2, ba2.reshape(1, HD), Wb2, bb2.reshape(1, HD), Wc2,
      bc2.reshape(1, 1))

    w2d = pl.pallas_call(
        _sel_body,
        out_shape=jax.ShapeDtypeStruct((ROWS, 128), f32),
    )(a1.reshape(ROWS, 128), a2.reshape(ROWS, 128),
      coords[:, 0].reshape(ROWS, 128), coords[:, 1].reshape(ROWS, 128))

    logits = pl.pallas_call(
        _pool_body,
        grid=(NBLK,),
        in_specs=[
            pl.BlockSpec((BLK, HD), lambda i: (i, 0)),
            pl.BlockSpec((BLK, 1), lambda i: (i, 0)),
            pl.BlockSpec((HD, 2), lambda i: (0, 0)),
            pl.BlockSpec((1, 2), lambda i: (0, 0)),
        ],
        out_specs=pl.BlockSpec((1, 2), lambda i: (0, 0)),
        out_shape=jax.ShapeDtypeStruct((1, 2), f32),
        scratch_shapes=[pltpu.VMEM((1, HD), f32)],
    )(x, w2d.reshape(N, 1), W_cls, b_cls.reshape(1, 2))

    return logits


# speedup vs baseline: 3.4396x; 3.4396x over previous
import functools

import jax
import jax.numpy as jnp
from jax.experimental import pallas as pl
from jax.experimental.pallas import tpu as pltpu

N = 32768
IN_DIM = 1024
HD = 256
K1 = 8
BLK = 1024
NBLK = N // BLK
ROWS = N // 128

def _dot(a, b):
    return jax.lax.dot_general(a.astype(jnp.bfloat16), b.astype(jnp.bfloat16),
                               (((1,), (0,)), ((), ())),
                               preferred_element_type=jnp.float32)


def _fwd_body(h_ref, wfc_ref, bfc_ref,
              wa1_ref, ba1_ref, wb1_ref, bb1_ref, wc1_ref, bc1_ref,
              wa2_ref, ba2_ref, wb2_ref, bb2_ref, wc2_ref, bc2_ref,
              x_ref, a1_ref, a2_ref):
    x = jnp.maximum(_dot(h_ref[:], wfc_ref[:]) + bfc_ref[:], 0.0)
    x_ref[:] = x

    def gated(wa, ba, wb, bb, wc, bc):
        a = jnp.tanh(_dot(x, wa) + ba)
        b = jax.nn.sigmoid(_dot(x, wb) + bb)
        return _dot(a * b, wc) + bc

    a1_ref[:] = gated(wa1_ref[:], ba1_ref[:], wb1_ref[:], bb1_ref[:],
                      wc1_ref[:], bc1_ref[:])
    a2_ref[:] = gated(wa2_ref[:], ba2_ref[:], wb2_ref[:], bb2_ref[:],
                      wc2_ref[:], bc2_ref[:])


def _sel_body(a1_ref, a2_ref, cx_ref, cy_ref, w_ref):
    rows = jax.lax.broadcasted_iota(jnp.int32, (ROWS, 128), 0)
    cols = jax.lax.broadcasted_iota(jnp.int32, (ROWS, 128), 1)
    flat = rows * 128 + cols

    a1 = a1_ref[:]
    cx = cx_ref[:]
    cy = cy_ref[:]
    big = jnp.float32(1e30)
    xmin = big
    xmax = -big
    ymin = big
    ymax = -big
    for _ in range(K1):
        m = jnp.max(a1)
        sidx = jnp.min(jnp.where(a1 >= m, flat, jnp.int32(2**30)))
        sel = flat == sidx
        xmin = jnp.minimum(xmin, jnp.min(jnp.where(sel, cx, big)))
        xmax = jnp.maximum(xmax, jnp.max(jnp.where(sel, cx, -big)))
        ymin = jnp.minimum(ymin, jnp.min(jnp.where(sel, cy, big)))
        ymax = jnp.maximum(ymax, jnp.max(jnp.where(sel, cy, -big)))
        a1 = jnp.where(sel, -big, a1)

    inb = ((cx >= xmin) & (cx <= xmax) & (cy >= ymin) & (cy <= ymax))
    a2 = a2_ref[:]
    mx = jnp.max(jnp.where(inb, a2, -big))
    e = jnp.where(inb, jnp.exp(a2 - mx), 0.0)
    z = jnp.sum(e)
    w_ref[:] = e / z


def _pool_body(x_ref, w_ref, wcls_ref, bcls_ref, out_ref, acc_ref):
    i = pl.program_id(0)

    @pl.when(i == 0)
    def _():
        acc_ref[:] = jnp.zeros_like(acc_ref)

    acc_ref[:] += jnp.sum(x_ref[:] * w_ref[:], axis=0, keepdims=True)

    @pl.when(i == NBLK - 1)
    def _():
        out_ref[:] = _dot(acc_ref[:], wcls_ref[:]) + bcls_ref[:]


@jax.jit
def kernel(h, coords, W_fc, b_fc, Wa1, ba1, Wb1, bb1, Wc1, bc1,
           Wa2, ba2, Wb2, bb2, Wc2, bc2, W_cls, b_cls):
    f32 = jnp.float32
    wspec = pl.BlockSpec((IN_DIM, HD), lambda i: (0, 0))
    hspec = pl.BlockSpec((HD, HD), lambda i: (0, 0))
    vspec = pl.BlockSpec((1, HD), lambda i: (0, 0))
    cspec = pl.BlockSpec((HD, 1), lambda i: (0, 0))
    sspec = pl.BlockSpec((1, 1), lambda i: (0, 0))

    x, a1, a2 = pl.pallas_call(
        _fwd_body,
        grid=(NBLK,),
        in_specs=[
            pl.BlockSpec((BLK, IN_DIM), lambda i: (i, 0)),
            wspec, vspec,
            hspec, vspec, hspec, vspec, cspec, sspec,
            hspec, vspec, hspec, vspec, cspec, sspec,
        ],
        out_specs=(
            pl.BlockSpec((BLK, HD), lambda i: (i, 0)),
            pl.BlockSpec((BLK, 1), lambda i: (i, 0)),
            pl.BlockSpec((BLK, 1), lambda i: (i, 0)),
        ),
        out_shape=(
            jax.ShapeDtypeStruct((N, HD), f32),
            jax.ShapeDtypeStruct((N, 1), f32),
            jax.ShapeDtypeStruct((N, 1), f32),
        ),
        compiler_params=pltpu.CompilerParams(
            dimension_semantics=("parallel",)),
    )(h, W_fc, b_fc.reshape(1, HD),
      Wa1, ba1.reshape(1, HD), Wb1, bb1.reshape(1, HD), Wc1,
      bc1.reshape(1, 1),
      Wa2, ba2.reshape(1, HD), Wb2, bb2.reshape(1, HD), Wc2,
      bc2.reshape(1, 1))

    w2d = pl.pallas_call(
        _sel_body,
        out_shape=jax.ShapeDtypeStruct((ROWS, 128), f32),
    )(a1.reshape(ROWS, 128), a2.reshape(ROWS, 128),
      coords[:, 0].reshape(ROWS, 128), coords[:, 1].reshape(ROWS, 128))

    logits = pl.pallas_call(
        _pool_body,
        grid=(NBLK,),
        in_specs=[
            pl.BlockSpec((BLK, HD), lambda i: (i, 0)),
            pl.BlockSpec((BLK, 1), lambda i: (i, 0)),
            pl.BlockSpec((HD, 2), lambda i: (0, 0)),
            pl.BlockSpec((1, 2), lambda i: (0, 0)),
        ],
        out_specs=pl.BlockSpec((1, 2), lambda i: (0, 0)),
        out_shape=jax.ShapeDtypeStruct((1, 2), f32),
        scratch_shapes=[pltpu.VMEM((1, HD), f32)],
    )(x, w2d.reshape(N, 1), W_cls, b_cls.reshape(1, 2))

    return logits


# speedup vs baseline: 3.4668x; 1.0079x over previous
import functools

import jax
import jax.numpy as jnp
from jax.experimental import pallas as pl
from jax.experimental.pallas import tpu as pltpu

N = 32768
IN_DIM = 1024
HD = 256
K1 = 8
BLK = 1024
NBLK = N // BLK
ROWS = N // 128

def _dot(a, b):
    return jax.lax.dot_general(a.astype(jnp.bfloat16), b.astype(jnp.bfloat16),
                               (((1,), (0,)), ((), ())),
                               preferred_element_type=jnp.float32)


def _fwd_body(h_ref, wfc_ref, bfc_ref,
              wa1_ref, ba1_ref, wb1_ref, bb1_ref, wc1_ref, bc1_ref,
              wa2_ref, ba2_ref, wb2_ref, bb2_ref, wc2_ref, bc2_ref,
              x_ref, a1_ref, a2_ref):
    x = jnp.maximum(_dot(h_ref[:], wfc_ref[:]) + bfc_ref[:], 0.0)
    x_ref[:] = x

    def gated(wa, ba, wb, bb, wc, bc):
        a = jnp.tanh(_dot(x, wa) + ba)
        b = jax.nn.sigmoid(_dot(x, wb) + bb)
        return _dot(a * b, wc) + bc

    a1_ref[:] = gated(wa1_ref[:], ba1_ref[:], wb1_ref[:], bb1_ref[:],
                      wc1_ref[:], bc1_ref[:])
    a2_ref[:] = gated(wa2_ref[:], ba2_ref[:], wb2_ref[:], bb2_ref[:],
                      wc2_ref[:], bc2_ref[:])


def _sel_body(a1_ref, a2_ref, cx_ref, cy_ref, w_ref):
    rows = jax.lax.broadcasted_iota(jnp.int32, (ROWS, 128), 0)
    cols = jax.lax.broadcasted_iota(jnp.int32, (ROWS, 128), 1)
    flat = rows * 128 + cols

    a1 = a1_ref[:]
    cx = cx_ref[:]
    cy = cy_ref[:]
    big = jnp.float32(1e30)
    xmin = big
    xmax = -big
    ymin = big
    ymax = -big
    for _ in range(K1):
        m = jnp.max(a1)
        sidx = jnp.min(jnp.where(a1 >= m, flat, jnp.int32(2**30)))
        sel = flat == sidx
        xmin = jnp.minimum(xmin, jnp.min(jnp.where(sel, cx, big)))
        xmax = jnp.maximum(xmax, jnp.max(jnp.where(sel, cx, -big)))
        ymin = jnp.minimum(ymin, jnp.min(jnp.where(sel, cy, big)))
        ymax = jnp.maximum(ymax, jnp.max(jnp.where(sel, cy, -big)))
        a1 = jnp.where(sel, -big, a1)

    inb = ((cx >= xmin) & (cx <= xmax) & (cy >= ymin) & (cy <= ymax))
    a2 = a2_ref[:]
    mx = jnp.max(jnp.where(inb, a2, -big))
    e = jnp.where(inb, jnp.exp(a2 - mx), 0.0)
    z = jnp.sum(e)
    w_ref[:] = e / z


def _pool_body(x_ref, w_ref, wcls_ref, bcls_ref, out_ref, acc_ref):
    i = pl.program_id(0)

    @pl.when(i == 0)
    def _():
        acc_ref[:] = jnp.zeros_like(acc_ref)

    acc_ref[:] += jnp.sum(x_ref[:] * w_ref[:], axis=0, keepdims=True)

    @pl.when(i == NBLK - 1)
    def _():
        out_ref[:] = _dot(acc_ref[:], wcls_ref[:]) + bcls_ref[:]


@jax.jit
def kernel(h, coords, W_fc, b_fc, Wa1, ba1, Wb1, bb1, Wc1, bc1,
           Wa2, ba2, Wb2, bb2, Wc2, bc2, W_cls, b_cls):
    f32 = jnp.float32
    wspec = pl.BlockSpec((IN_DIM, HD), lambda i: (0, 0))
    hspec = pl.BlockSpec((HD, HD), lambda i: (0, 0))
    vspec = pl.BlockSpec((1, HD), lambda i: (0, 0))
    cspec = pl.BlockSpec((HD, 1), lambda i: (0, 0))
    sspec = pl.BlockSpec((1, 1), lambda i: (0, 0))

    x, a1, a2 = pl.pallas_call(
        _fwd_body,
        grid=(NBLK,),
        in_specs=[
            pl.BlockSpec((BLK, IN_DIM), lambda i: (i, 0)),
            wspec, vspec,
            hspec, vspec, hspec, vspec, cspec, sspec,
            hspec, vspec, hspec, vspec, cspec, sspec,
        ],
        out_specs=(
            pl.BlockSpec((BLK, HD), lambda i: (i, 0)),
            pl.BlockSpec((BLK, 1), lambda i: (i, 0)),
            pl.BlockSpec((BLK, 1), lambda i: (i, 0)),
        ),
        out_shape=(
            jax.ShapeDtypeStruct((N, HD), f32),
            jax.ShapeDtypeStruct((N, 1), f32),
            jax.ShapeDtypeStruct((N, 1), f32),
        ),
        compiler_params=pltpu.CompilerParams(
            dimension_semantics=("arbitrary",)),
    )(h, W_fc, b_fc.reshape(1, HD),
      Wa1, ba1.reshape(1, HD), Wb1, bb1.reshape(1, HD), Wc1,
      bc1.reshape(1, 1),
      Wa2, ba2.reshape(1, HD), Wb2, bb2.reshape(1, HD), Wc2,
      bc2.reshape(1, 1))

    w2d = pl.pallas_call(
        _sel_body,
        out_shape=jax.ShapeDtypeStruct((ROWS, 128), f32),
    )(a1.reshape(ROWS, 128), a2.reshape(ROWS, 128),
      coords[:, 0].reshape(ROWS, 128), coords[:, 1].reshape(ROWS, 128))

    logits = pl.pallas_call(
        _pool_body,
        grid=(NBLK,),
        in_specs=[
            pl.BlockSpec((BLK, HD), lambda i: (i, 0)),
            pl.BlockSpec((BLK, 1), lambda i: (i, 0)),
            pl.BlockSpec((HD, 2), lambda i: (0, 0)),
            pl.BlockSpec((1, 2), lambda i: (0, 0)),
        ],
        out_specs=pl.BlockSpec((1, 2), lambda i: (0, 0)),
        out_shape=jax.ShapeDtypeStruct((1, 2), f32),
        scratch_shapes=[pltpu.VMEM((1, HD), f32)],
    )(x, w2d.reshape(N, 1), W_cls, b_cls.reshape(1, 2))

    return logits


# speedup vs baseline: 5.2465x; 1.5134x over previous
import jax
import jax.numpy as jnp
from jax.experimental import pallas as pl
from jax.experimental.pallas import tpu as pltpu

N = 32768
IN_DIM = 1024
HD = 256
K1 = 8
BLK = 1024
NBLK = N // BLK


def _dot(a, b):
    return jax.lax.dot_general(a.astype(jnp.bfloat16), b.astype(jnp.bfloat16),
                               (((1,), (0,)), ((), ())),
                               preferred_element_type=jnp.float32)


def _body(cx_ref, cy_ref, h_ref, wfc_ref, bfc_ref,
          wa1_ref, ba1_ref, wb1_ref, bb1_ref, wc1_ref, bc1_ref,
          wa2_ref, ba2_ref, wb2_ref, bb2_ref, wc2_ref, bc2_ref,
          wcls_ref, bcls_ref,
          out_ref, x_scr, a1_scr, a2_scr):
    i = pl.program_id(0)
    x = jnp.maximum(_dot(h_ref[:], wfc_ref[:]) + bfc_ref[:], 0.0)
    xb = x.astype(jnp.bfloat16)
    x_scr[pl.ds(i * BLK, BLK), :] = xb

    def gated(wa, ba, wb, bb, wc, bc):
        a = jnp.tanh(_dot(xb, wa) + ba)
        b = 0.5 * jnp.tanh(0.5 * (_dot(xb, wb) + bb)) + 0.5
        return _dot(a * b, wc) + bc

    a1 = gated(wa1_ref[:], ba1_ref[:], wb1_ref[:], bb1_ref[:],
               wc1_ref[:], bc1_ref[:])
    a2 = gated(wa2_ref[:], ba2_ref[:], wb2_ref[:], bb2_ref[:],
               wc2_ref[:], bc2_ref[:])
    a1_scr[pl.ds(i, 1), :] = jnp.transpose(a1, (1, 0))
    a2_scr[pl.ds(i, 1), :] = jnp.transpose(a2, (1, 0))

    @pl.when(i == NBLK - 1)
    def _():
        rows = jax.lax.broadcasted_iota(jnp.int32, (NBLK, BLK), 0)
        cols = jax.lax.broadcasted_iota(jnp.int32, (NBLK, BLK), 1)
        flat = rows * BLK + cols

        a1v = a1_scr[:]
        cx = cx_ref[:]
        cy = cy_ref[:]
        big = jnp.float32(1e30)
        xmin = big
        xmax = -big
        ymin = big
        ymax = -big
        for _ in range(K1):
            m = jnp.max(a1v)
            sidx = jnp.min(jnp.where(a1v >= m, flat, jnp.int32(2**30)))
            sel = flat == sidx
            xmin = jnp.minimum(xmin, jnp.min(jnp.where(sel, cx, big)))
            xmax = jnp.maximum(xmax, jnp.max(jnp.where(sel, cx, -big)))
            ymin = jnp.minimum(ymin, jnp.min(jnp.where(sel, cy, big)))
            ymax = jnp.maximum(ymax, jnp.max(jnp.where(sel, cy, -big)))
            a1v = jnp.where(sel, -big, a1v)

        inb = ((cx >= xmin) & (cx <= xmax) & (cy >= ymin) & (cy <= ymax))
        a2v = a2_scr[:]
        mx = jnp.max(jnp.where(inb, a2v, -big))
        e = jnp.where(inb, jnp.exp(a2v - mx), 0.0)
        z = jnp.sum(e)
        w = (e / z).astype(jnp.bfloat16)

        acc = jnp.zeros((1, HD), jnp.float32)
        for j in range(NBLK):
            wj = w[j:j + 1, :]
            acc += jax.lax.dot_general(
                wj, x_scr[pl.ds(j * BLK, BLK), :],
                (((1,), (0,)), ((), ())),
                preferred_element_type=jnp.float32)
        out_ref[:] = _dot(acc, wcls_ref[:]) + bcls_ref[:]


@jax.jit
def kernel(h, coords, W_fc, b_fc, Wa1, ba1, Wb1, bb1, Wc1, bc1,
           Wa2, ba2, Wb2, bb2, Wc2, bc2, W_cls, b_cls):
    f32 = jnp.float32
    full = lambda r, c: pl.BlockSpec((r, c), lambda i: (0, 0))

    logits = pl.pallas_call(
        _body,
        grid=(NBLK,),
        in_specs=[
            full(NBLK, BLK), full(NBLK, BLK),
            pl.BlockSpec((BLK, IN_DIM), lambda i: (i, 0)),
            full(IN_DIM, HD), full(1, HD),
            full(HD, HD), full(1, HD), full(HD, HD), full(1, HD),
            full(HD, 1), full(1, 1),
            full(HD, HD), full(1, HD), full(HD, HD), full(1, HD),
            full(HD, 1), full(1, 1),
            full(HD, 2), full(1, 2),
        ],
        out_specs=pl.BlockSpec((1, 2), lambda i: (0, 0)),
        out_shape=jax.ShapeDtypeStruct((1, 2), f32),
        scratch_shapes=[
            pltpu.VMEM((N, HD), jnp.bfloat16),
            pltpu.VMEM((NBLK, BLK), f32),
            pltpu.VMEM((NBLK, BLK), f32),
        ],
        compiler_params=pltpu.CompilerParams(
            dimension_semantics=("arbitrary",)),
    )(coords[:, 0].reshape(NBLK, BLK), coords[:, 1].reshape(NBLK, BLK),
      h, W_fc, b_fc.reshape(1, HD),
      Wa1, ba1.reshape(1, HD), Wb1, bb1.reshape(1, HD),
      Wc1, bc1.reshape(1, 1),
      Wa2, ba2.reshape(1, HD), Wb2, bb2.reshape(1, HD),
      Wc2, bc2.reshape(1, 1),
      W_cls, b_cls.reshape(1, 2))

    return logits


# speedup vs baseline: 5.7545x; 1.0968x over previous
import jax
import jax.numpy as jnp
from jax.experimental import pallas as pl
from jax.experimental.pallas import tpu as pltpu

N = 32768
IN_DIM = 1024
HD = 256
K1 = 8
BLK = 1024
NBLK = N // BLK


def _dot(a, b):
    return jax.lax.dot_general(a.astype(jnp.bfloat16), b.astype(jnp.bfloat16),
                               (((1,), (0,)), ((), ())),
                               preferred_element_type=jnp.float32)


def _body(cx_ref, cy_ref, h_ref, wfc_ref, bfc_ref,
          wab_ref, bab_ref, wc_ref, bc_ref,
          wcls_ref, bcls_ref,
          out_ref, x_scr, a_scr):
    i = pl.program_id(0)
    x = jnp.maximum(_dot(h_ref[:], wfc_ref[:]) + bfc_ref[:], 0.0)
    xb = x.astype(jnp.bfloat16)
    x_scr[pl.ds(i * BLK, BLK), :] = xb

    t = jnp.tanh(_dot(xb, wab_ref[:]) + bab_ref[:])
    g1 = t[:, 0 * HD:1 * HD] * (0.5 * t[:, 1 * HD:2 * HD] + 0.5)
    g2 = t[:, 2 * HD:3 * HD] * (0.5 * t[:, 3 * HD:4 * HD] + 0.5)
    a12 = _dot(jnp.concatenate([g1, g2], axis=1), wc_ref[:]) + bc_ref[:]
    at = jnp.transpose(a12, (1, 0))
    a_scr[pl.ds(i, 1), :] = at[0:1, :]
    a_scr[pl.ds(NBLK + i, 1), :] = at[1:2, :]

    @pl.when(i == NBLK - 1)
    def _():
        rows = jax.lax.broadcasted_iota(jnp.int32, (NBLK, BLK), 0)
        cols = jax.lax.broadcasted_iota(jnp.int32, (NBLK, BLK), 1)
        flat = rows * BLK + cols

        a1v = a_scr[0:NBLK, :]
        cx = cx_ref[:]
        cy = cy_ref[:]
        big = jnp.float32(1e30)
        xmin = big
        xmax = -big
        ymin = big
        ymax = -big
        for _ in range(K1):
            m = jnp.max(a1v)
            sidx = jnp.min(jnp.where(a1v >= m, flat, jnp.int32(2**30)))
            sel = flat == sidx
            xmin = jnp.minimum(xmin, jnp.min(jnp.where(sel, cx, big)))
            xmax = jnp.maximum(xmax, jnp.max(jnp.where(sel, cx, -big)))
            ymin = jnp.minimum(ymin, jnp.min(jnp.where(sel, cy, big)))
            ymax = jnp.maximum(ymax, jnp.max(jnp.where(sel, cy, -big)))
            a1v = jnp.where(sel, -big, a1v)

        inb = ((cx >= xmin) & (cx <= xmax) & (cy >= ymin) & (cy <= ymax))
        a2v = a_scr[NBLK:2 * NBLK, :]
        mx = jnp.max(jnp.where(inb, a2v, -big))
        e = jnp.where(inb, jnp.exp(a2v - mx), 0.0)
        z = jnp.sum(e)
        w = (e / z).astype(jnp.bfloat16)

        acc = jnp.zeros((1, HD), jnp.float32)
        for j in range(NBLK):
            acc += jax.lax.dot_general(
                w[j:j + 1, :], x_scr[pl.ds(j * BLK, BLK), :],
                (((1,), (0,)), ((), ())),
                preferred_element_type=jnp.float32)
        out_ref[:] = _dot(acc, wcls_ref[:]) + bcls_ref[:]


@jax.jit
def kernel(h, coords, W_fc, b_fc, Wa1, ba1, Wb1, bb1, Wc1, bc1,
           Wa2, ba2, Wb2, bb2, Wc2, bc2, W_cls, b_cls):
    f32 = jnp.float32
    full = lambda r, c: pl.BlockSpec((r, c), lambda i: (0, 0))

    Wab = jnp.concatenate([Wa1, 0.5 * Wb1, Wa2, 0.5 * Wb2], axis=1)
    bab = jnp.concatenate([ba1, 0.5 * bb1, ba2, 0.5 * bb2]).reshape(1, 4 * HD)
    z = jnp.zeros((HD, 1), f32)
    Wc = jnp.concatenate(
        [jnp.concatenate([Wc1, z], axis=1),
         jnp.concatenate([z, Wc2], axis=1)], axis=0)
    bc = jnp.stack([bc1[0], bc2[0]]).reshape(1, 2)

    logits = pl.pallas_call(
        _body,
        grid=(NBLK,),
        in_specs=[
            full(NBLK, BLK), full(NBLK, BLK),
            pl.BlockSpec((BLK, IN_DIM), lambda i: (i, 0)),
            full(IN_DIM, HD), full(1, HD),
            full(HD, 4 * HD), full(1, 4 * HD),
            full(2 * HD, 2), full(1, 2),
            full(HD, 2), full(1, 2),
        ],
        out_specs=pl.BlockSpec((1, 2), lambda i: (0, 0)),
        out_shape=jax.ShapeDtypeStruct((1, 2), f32),
        scratch_shapes=[
            pltpu.VMEM((N, HD), jnp.bfloat16),
            pltpu.VMEM((2 * NBLK, BLK), f32),
        ],
        compiler_params=pltpu.CompilerParams(
            dimension_semantics=("arbitrary",)),
    )(coords[:, 0].reshape(NBLK, BLK), coords[:, 1].reshape(NBLK, BLK),
      h, W_fc, b_fc.reshape(1, HD),
      Wab, bab, Wc, bc,
      W_cls, b_cls.reshape(1, 2))

    return logits
